# Initial kernel scaffold; baseline (speedup 1.0000x reference)
import jax
import jax.numpy as jnp
from jax.experimental import pallas as pl


def kernel(total_scores, eliminated_idx_list, mask):
    raise NotImplementedError("write your pallas kernel here")


# speedup vs baseline: 1.9316x; 1.9316x over previous
import functools

import jax
import jax.numpy as jnp
from jax import lax
from jax.experimental import pallas as pl
from jax.experimental.pallas import tpu as pltpu
from jax.experimental.pallas import tpu_sc as plsc

_MARGIN = 0.01
_BIG = 1e30

_B, _N, _K = 16, 2048, 256
_NC, _NS, _L = 2, 16, 16
_NW = _NC * _NS
_HALF = _K // _NC
_NV = _N // _L
_KV = _HALF // _L
_NACC = 4


def _worker_body(scores_hbm, maskf_hbm, idx_hbm, ihalf_hbm,
                 out_hbm,
                 s_v, m_v, il_v, sp_v, idx_v, ih_v, t_v, o_v):
    c = lax.axis_index("c")
    s = lax.axis_index("s")
    wid = s * _NC + c
    row = s
    half = c

    pltpu.sync_copy(scores_hbm.at[pl.ds(row * _N, _N)], s_v)
    pltpu.sync_copy(maskf_hbm.at[pl.ds(row * _N, _N)], m_v)
    pltpu.sync_copy(idx_hbm.at[pl.ds(row * _K, _K)], idx_v)
    pltpu.sync_copy(ihalf_hbm.at[pl.ds(row * _K + half * _HALF, _HALF)], ih_v)

    zeros = jnp.zeros((_L,), jnp.float32)
    ones = jnp.ones((_L,), jnp.float32)

    for i in range(_NV):
        il_v[pl.ds(i * _L, _L)] = zeros
    for j in range(_K // _L):
        iv = idx_v[pl.ds(j * _L, _L)]
        plsc.store_scatter(il_v, [iv], ones)

    scnt = zeros
    for i in range(_NV):
        sl = s_v[pl.ds(i * _L, _L)]
        ml = m_v[pl.ds(i * _L, _L)]
        mark = il_v[pl.ds(i * _L, _L)]
        surv = (ml > 0.0) & (mark == 0.0)
        sp_v[pl.ds(i * _L, _L)] = jnp.where(surv, sl, _BIG)
        scnt = scnt + jnp.where(surv, 1.0, 0.0)

    ecnt = zeros
    for j in range(_KV):
        eidx = ih_v[pl.ds(j * _L, _L)]
        es = plsc.load_gather(s_v, [eidx])
        ew = plsc.load_gather(m_v, [eidx])
        valid = ew > 0.0
        t_v[pl.ds(j * _L, _L)] = jnp.where(valid, es + _MARGIN, -_BIG)
        ecnt = ecnt + jnp.where(valid, 1.0, 0.0)

    def jbody(j, accs):
        tvec = t_v[pl.ds(j * _L, _L)]
        ts = [tvec[l] for l in range(_L)]

        def ibody(i, iaccs):
            iaccs = list(iaccs)
            sp = sp_v[pl.ds(i * _L, _L)]
            for l in range(_L):
                iaccs[l % _NACC] = iaccs[l % _NACC] + jnp.maximum(ts[l] - sp, 0.0)
            return tuple(iaccs)

        return lax.fori_loop(0, _NV, ibody, accs, unroll=4)

    accs = lax.fori_loop(0, _KV, jbody, (zeros,) * _NACC)
    acc = accs[0] + accs[1] + accs[2] + accs[3]

    loss = jnp.sum(acc)
    pairs = jnp.sum(ecnt) * jnp.sum(scnt)
    lane = lax.iota(jnp.int32, _L)
    o_v[...] = jnp.where(lane == 0, loss, jnp.where(lane == 1, pairs, 0.0))
    pltpu.sync_copy(o_v, out_hbm.at[pl.ds(wid * _L, _L)])


_sc_call = functools.partial(
    pl.kernel,
    out_type=jax.ShapeDtypeStruct((_NW * _L,), jnp.float32),
    mesh=plsc.VectorSubcoreMesh(core_axis_name="c", subcore_axis_name="s"),
    compiler_params=pltpu.CompilerParams(needs_layout_passes=False),
    scratch_types=[
        pltpu.VMEM((_N,), jnp.float32),
        pltpu.VMEM((_N,), jnp.float32),
        pltpu.VMEM((_N,), jnp.float32),
        pltpu.VMEM((_N,), jnp.float32),
        pltpu.VMEM((_K,), jnp.int32),
        pltpu.VMEM((_HALF,), jnp.int32),
        pltpu.VMEM((_HALF,), jnp.float32),
        pltpu.VMEM((_L,), jnp.float32),
    ],
)(_worker_body)


def kernel(total_scores, eliminated_idx_list, mask):
    scores_flat = total_scores.reshape(-1)
    maskf_flat = mask.astype(jnp.float32).reshape(-1)
    idx_flat = eliminated_idx_list.reshape(-1)
    out = _sc_call(scores_flat, maskf_flat, idx_flat, idx_flat)
    part = out.reshape(_NW, _L)
    total_loss = part[:, 0].sum()
    total_pairs = part[:, 1].sum()
    return jnp.where(total_pairs > 0, total_loss / total_pairs, total_loss)


# speedup vs baseline: 2.3828x; 1.2336x over previous
import functools

import jax
import jax.numpy as jnp
from jax import lax
from jax.experimental import pallas as pl
from jax.experimental.pallas import tpu as pltpu
from jax.experimental.pallas import tpu_sc as plsc

_MARGIN = 0.01
_BIG = 1e30

_B, _N, _K = 16, 2048, 256
_NC, _NS, _L = 2, 16, 16
_NW = _NC * _NS
_HALF = _K // _NC
_NV = _N // _L
_KV = _HALF // _L
_UNR = 4
_SP_PAD = _N + _UNR * _L
_T_PAD = _HALF + _L


def _worker_body(scores_hbm, maskf_hbm, idx_hbm,
                 out_hbm,
                 s_v, m_v, il_v, sp_v, idx_v, t_v, o_v):
    c = lax.axis_index("c")
    s = lax.axis_index("s")
    wid = s * _NC + c
    row = s
    half = c

    pltpu.sync_copy(scores_hbm.at[pl.ds(row * _N, _N)], s_v)
    pltpu.sync_copy(maskf_hbm.at[pl.ds(row * _N, _N)], m_v)
    pltpu.sync_copy(idx_hbm.at[pl.ds(row * _K, _K)], idx_v)

    zeros = jnp.zeros((_L,), jnp.float32)
    ones = jnp.ones((_L,), jnp.float32)
    bigs = jnp.full((_L,), _BIG, jnp.float32)
    nbigs = jnp.full((_L,), -_BIG, jnp.float32)

    for i in range(_NV):
        il_v[pl.ds(i * _L, _L)] = zeros
    for i in range(_SP_PAD // _L):
        sp_v[pl.ds(i * _L, _L)] = bigs
    for j in range(_T_PAD // _L):
        t_v[pl.ds(j * _L, _L)] = nbigs
    for j in range(_K // _L):
        iv = idx_v[pl.ds(j * _L, _L)]
        plsc.store_scatter(il_v, [iv], ones)

    scnt = jnp.int32(0)
    for i in range(_NV):
        sl = s_v[pl.ds(i * _L, _L)]
        ml = m_v[pl.ds(i * _L, _L)]
        mark = il_v[pl.ds(i * _L, _L)]
        surv = (ml > 0.0) & (mark == 0.0)
        plsc.store_compressed(sp_v.at[pl.ds(scnt, _L)], sl, mask=surv)
        scnt = scnt + plsc.all_reduce_population_count(surv)[0]

    ecnt = jnp.int32(0)
    for j in range(_KV):
        eidx = idx_v[pl.ds(half * _HALF + j * _L, _L)]
        es = plsc.load_gather(s_v, [eidx])
        ew = plsc.load_gather(m_v, [eidx])
        valid = ew > 0.0
        plsc.store_compressed(t_v.at[pl.ds(ecnt, _L)], es + _MARGIN, mask=valid)
        ecnt = ecnt + plsc.all_reduce_population_count(valid)[0]

    kv = (ecnt + _L - 1) // _L
    nv = (scnt + _UNR * _L - 1) // (_UNR * _L)

    def kbody(r, accs):
        tvec = t_v[pl.ds(r * _L, _L)]
        ts = [tvec[l] for l in range(_L)]

        def ibody(i, iaccs):
            iaccs = list(iaccs)
            for q in range(_UNR):
                sp = sp_v[pl.ds(i * (_UNR * _L) + q * _L, _L)]
                for l in range(_L):
                    a = (q * _L + l) % _UNR
                    iaccs[a] = iaccs[a] + jnp.maximum(ts[l] - sp, 0.0)
            return tuple(iaccs)

        return lax.fori_loop(0, nv, ibody, accs)

    accs = lax.fori_loop(0, kv, kbody, (zeros,) * _UNR)
    acc = accs[0] + accs[1] + accs[2] + accs[3]

    loss = jnp.sum(acc)
    pairs = ecnt.astype(jnp.float32) * scnt.astype(jnp.float32)
    lane = lax.iota(jnp.int32, _L)
    o_v[...] = jnp.where(lane == 0, loss, jnp.where(lane == 1, pairs, 0.0))
    pltpu.sync_copy(o_v, out_hbm.at[pl.ds(wid * _L, _L)])


_sc_call = functools.partial(
    pl.kernel,
    out_type=jax.ShapeDtypeStruct((_NW * _L,), jnp.float32),
    mesh=plsc.VectorSubcoreMesh(core_axis_name="c", subcore_axis_name="s"),
    compiler_params=pltpu.CompilerParams(needs_layout_passes=False),
    scratch_types=[
        pltpu.VMEM((_N,), jnp.float32),
        pltpu.VMEM((_N,), jnp.float32),
        pltpu.VMEM((_N,), jnp.float32),
        pltpu.VMEM((_SP_PAD,), jnp.float32),
        pltpu.VMEM((_K,), jnp.int32),
        pltpu.VMEM((_T_PAD,), jnp.float32),
        pltpu.VMEM((_L,), jnp.float32),
    ],
)(_worker_body)


def kernel(total_scores, eliminated_idx_list, mask):
    scores_flat = total_scores.reshape(-1)
    maskf_flat = mask.astype(jnp.float32).reshape(-1)
    idx_flat = eliminated_idx_list.reshape(-1)
    out = _sc_call(scores_flat, maskf_flat, idx_flat)
    part = out.reshape(_NW, _L)
    total_loss = part[:, 0].sum()
    total_pairs = part[:, 1].sum()
    return jnp.where(total_pairs > 0, total_loss / total_pairs, total_loss)
